# baseline (speedup 1.0000x reference)
import functools

import jax
import jax.numpy as jnp
from jax import lax
from jax.experimental import pallas as pl
from jax.experimental.pallas import tpu as pltpu
from jax.experimental.pallas import tpu_sc as plsc


_L = 16
_BW = 128
_TP = 129


@functools.lru_cache(maxsize=None)
def _make_retile(num_rows: int, feat: int):
    info = plsc.get_sparse_core_info()
    nc, ns = info.num_cores, info.num_subcores
    nw = nc * ns
    assert feat == 32
    n_full = num_rows // _BW
    tail = num_rows - n_full * _BW
    per_w = n_full // nw
    n_extra = n_full - per_w * nw
    assert per_w % 2 == 0 and tail % 4 == 0 and n_extra < nw
    mesh = plsc.VectorSubcoreMesh(core_axis_name="core", subcore_axis_name="sub")

    @functools.partial(
        pl.kernel,
        out_type=jax.ShapeDtypeStruct((num_rows // 4, _BW), jnp.float32),
        mesh=mesh,
        scratch_types=[
            [pltpu.VMEM((feat, _TP), jnp.float32)] * 2,
            [pltpu.VMEM((feat, _BW), jnp.float32)] * 2,
            [pltpu.SemaphoreType.DMA] * 2,
            [pltpu.SemaphoreType.DMA] * 2,
        ],
        compiler_params=pltpu.CompilerParams(needs_layout_passes=False),
    )
    def retile_kernel(tab_hbm, out_hbm, in_v, o_v, sems_g, sems_s):
        w = lax.axis_index("sub") * nc + lax.axis_index("core")
        base = w * per_w

        iota = lax.iota(jnp.int32, _L)
        f_lo = iota
        f_hi = iota + _L

        def fire(j, b):
            return [
                pltpu.async_copy(
                    tab_hbm.at[i, :, pl.ds(j * _BW, _BW)],
                    in_v[b].at[pl.ds(i * 8, 8), pl.ds(0, _BW)],
                    sems_g[b],
                )
                for i in range(feat // 8)
            ]

        def transpose(b, width):
            def tbody(a, carry):
                for q in range(4):
                    c = a * 4 + q
                    cv = jnp.full((_L,), c, dtype=jnp.int32)
                    o_v[b][a, pl.ds(q * feat, _L)] = plsc.load_gather(
                        in_v[b], [f_lo, cv])
                    o_v[b][a, pl.ds(q * feat + _L, _L)] = plsc.load_gather(
                        in_v[b], [f_hi, cv])
                return carry
            lax.fori_loop(0, width // 4, tbody, 0)

        def store(j, b, width):
            rows = width // 4
            return pltpu.async_copy(
                o_v[b].at[pl.ds(0, rows)],
                out_hbm.at[pl.ds(j * (_BW // 4), rows)],
                sems_s[b],
            )

        def body(p2, carry):
            j0 = base + p2 * 2
            g0 = fire(j0, 0)
            g1 = fire(j0 + 1, 1)
            for cp in g0:
                cp.wait()
            transpose(0, _BW)
            s0 = store(j0, 0, _BW)
            for cp in g1:
                cp.wait()
            transpose(1, _BW)
            s1 = store(j0 + 1, 1, _BW)
            s0.wait()
            s1.wait()
            return carry

        lax.fori_loop(0, per_w // 2, body, 0)

        @pl.when(w < n_extra)
        def _():
            j = n_full - n_extra + w
            for cp in fire(j, 0):
                cp.wait()
            transpose(0, _BW)
            store(j, 0, _BW).wait()

        if tail:
            @pl.when(w == n_extra)
            def _():
                for i in range(feat // 8):
                    for s in range(8):
                        pltpu.sync_copy(
                            tab_hbm.at[i, s, pl.ds(n_full * _BW, tail)],
                            in_v[0].at[i * 8 + s, pl.ds(0, tail)],
                        )
                transpose(0, tail)
                store(n_full, 0, tail).wait()

    def run(table):
        tab3 = jnp.swapaxes(table, 0, 1).reshape(4, 8, num_rows)
        return retile_kernel(tab3)

    return run


@functools.lru_cache(maxsize=None)
def _make_gather(num_rows: int, feat: int, nb: int, nh: int):
    info = plsc.get_sparse_core_info()
    nc, ns = info.num_cores, info.num_subcores
    nw = nc * ns
    assert nb == nw * _BW and feat == 32 and nh % 2 == 0
    ftiles = feat // 8
    mesh = plsc.VectorSubcoreMesh(core_axis_name="core", subcore_axis_name="sub")

    @functools.partial(
        pl.kernel,
        out_type=jax.ShapeDtypeStruct((nh * ftiles * nw * 8, _BW), jnp.float32),
        mesh=mesh,
        scratch_types=[
            pltpu.VMEM((nh, _BW), jnp.int32),
            [pltpu.VMEM((_BW, feat), jnp.float32)] * 2,
            [pltpu.VMEM((feat, _TP), jnp.float32)] * 2,
            [pltpu.SemaphoreType.DMA] * 2,
            [pltpu.SemaphoreType.DMA] * 2,
        ],
        compiler_params=pltpu.CompilerParams(
            use_tc_tiling_on_sc=False, needs_layout_passes=False),
    )
    def gather_kernel(idx_hbm, table_hbm, out_hbm, idx_v, rows_v, t_v,
                      sems_g, sems_s):
        w = lax.axis_index("sub") * nc + lax.axis_index("core")
        pltpu.sync_copy(idx_hbm.at[:, pl.ds(w * _BW, _BW)], idx_v)

        iota = lax.iota(jnp.int32, _L)
        f_lo = iota
        f_hi = iota + _L

        def fire(h, b):
            return pltpu.async_copy(
                table_hbm.at[idx_v.at[h]], rows_v[b], sems_g[b])

        def transpose(b):
            def tbody(l4, carry):
                for k in range(4):
                    l = l4 * 4 + k
                    lv = jnp.full((_L,), l, dtype=jnp.int32)
                    v1 = rows_v[b][l, pl.ds(0, _L)]
                    v2 = rows_v[b][l, pl.ds(_L, _L)]
                    plsc.store_scatter(t_v[b], [f_lo, lv], v1)
                    plsc.store_scatter(t_v[b], [f_hi, lv], v2)
                return carry
            lax.fori_loop(0, _BW // 4, tbody, 0)

        def store(h, b):
            return [
                pltpu.async_copy(
                    t_v[b].at[pl.ds(i * 8, 8), pl.ds(0, _BW)],
                    out_hbm.at[pl.ds((((h * ftiles) + i) * nw + w) * 8, 8)],
                    sems_s[b],
                )
                for i in range(ftiles)
            ]

        def body(p, carry):
            h0 = p * 2
            g0 = fire(h0, 0)
            g1 = fire(h0 + 1, 1)
            g0.wait()
            transpose(0)
            s0 = store(h0, 0)
            g1.wait()
            transpose(1)
            s1 = store(h0 + 1, 1)
            for cp in s0 + s1:
                cp.wait()
            return carry

        lax.fori_loop(0, nh // 2, body, 0)

    def run(x, table_flat):
        idx_hm = jnp.transpose(x, (1, 2, 0)).reshape(nh, nb)
        out = gather_kernel(idx_hm, table_flat)
        out5 = out.reshape(nh, ftiles, nw, 8, _BW)
        return jnp.transpose(out5, (2, 4, 0, 1, 3)).reshape(nb, nh, feat)

    return run


def kernel(x, table):
    b, h = x.shape[0], x.shape[1]
    v, feat = table.shape
    retile = _make_retile(v, feat)
    table_flat = retile(table).reshape(v, feat)
    run = _make_gather(v, feat, b, h)
    return run(x, table_flat)


# speedup vs baseline: 1.5540x; 1.5540x over previous
import functools

import jax
import jax.numpy as jnp
from jax import lax
from jax.experimental import pallas as pl
from jax.experimental.pallas import tpu as pltpu
from jax.experimental.pallas import tpu_sc as plsc


_L = 16
_BW = 128
_TP = 129


@functools.lru_cache(maxsize=None)
def _make_gather(num_rows: int, feat: int, nb: int, nh: int):
    info = plsc.get_sparse_core_info()
    nc, ns = info.num_cores, info.num_subcores
    nw = nc * ns
    assert nb == nw * _BW and feat == 32 and nh % 2 == 0
    ftiles = feat // 8
    mesh = plsc.VectorSubcoreMesh(core_axis_name="core", subcore_axis_name="sub")

    @functools.partial(
        pl.kernel,
        out_type=jax.ShapeDtypeStruct((nh * ftiles * nw * 8, _BW), jnp.float32),
        mesh=mesh,
        scratch_types=[
            pltpu.VMEM((nh, _BW), jnp.int32),
            [pltpu.VMEM((_BW, feat), jnp.float32)] * 2,
            [pltpu.VMEM((feat, _TP), jnp.float32)] * 2,
            [pltpu.SemaphoreType.DMA] * 2,
            [pltpu.SemaphoreType.DMA] * 2,
        ],
        compiler_params=pltpu.CompilerParams(
            use_tc_tiling_on_sc=False, needs_layout_passes=False),
    )
    def gather_kernel(idx_hbm, table_hbm, out_hbm, idx_v, rows_v, t_v,
                      sems_g, sems_s):
        w = lax.axis_index("sub") * nc + lax.axis_index("core")
        pltpu.sync_copy(idx_hbm.at[:, pl.ds(w * _BW, _BW)], idx_v)

        iota = lax.iota(jnp.int32, _L)
        f_lo = iota
        f_hi = iota + _L

        def fire(h, b):
            return pltpu.async_copy(
                table_hbm.at[idx_v.at[h]], rows_v[b], sems_g[b])

        def transpose(b):
            def tbody(l4, carry):
                for k in range(4):
                    l = l4 * 4 + k
                    lv = jnp.full((_L,), l, dtype=jnp.int32)
                    v1 = rows_v[b][l, pl.ds(0, _L)]
                    v2 = rows_v[b][l, pl.ds(_L, _L)]
                    plsc.store_scatter(t_v[b], [f_lo, lv], v1)
                    plsc.store_scatter(t_v[b], [f_hi, lv], v2)
                return carry
            lax.fori_loop(0, _BW // 4, tbody, 0)

        def store(h, b):
            return [
                pltpu.async_copy(
                    t_v[b].at[pl.ds(i * 8, 8), pl.ds(0, _BW)],
                    out_hbm.at[pl.ds((((h * ftiles) + i) * nw + w) * 8, 8)],
                    sems_s[b],
                )
                for i in range(ftiles)
            ]

        def body(p, carry):
            h0 = p * 2
            g0 = fire(h0, 0)
            g1 = fire(h0 + 1, 1)
            g0.wait()
            transpose(0)
            s0 = store(h0, 0)
            g1.wait()
            transpose(1)
            s1 = store(h0 + 1, 1)
            for cp in s0 + s1:
                cp.wait()
            return carry

        lax.fori_loop(0, nh // 2, body, 0)

    def run(x, table):
        idx_hm = jnp.transpose(x, (1, 2, 0)).reshape(nh, nb)
        out = gather_kernel(idx_hm, table)
        out5 = out.reshape(nh, ftiles, nw, 8, _BW)
        return jnp.transpose(out5, (2, 4, 0, 1, 3)).reshape(nb, nh, feat)

    return run


def kernel(x, table):
    b, h = x.shape[0], x.shape[1]
    run = _make_gather(table.shape[0], table.shape[1], b, h)
    return run(x, table)
